# Initial kernel scaffold; baseline (speedup 1.0000x reference)
import jax
import jax.numpy as jnp
from jax.experimental import pallas as pl


def kernel(x, hx, edge_index, W0_r_x, W1_r_x, b_r_x, W0_r_h, W1_r_h, b_r_h, W0_u_x, W1_u_x, b_u_x, W0_u_h, W1_u_h, b_u_h, W0_c_x, W1_c_x, b_c_x, W0_c_h, W1_c_h, b_c_h):
    raise NotImplementedError("write your pallas kernel here")


# speedup vs baseline: 3.9274x; 3.9274x over previous
import functools

import jax
import jax.numpy as jnp
from jax import lax
from jax.experimental import pallas as pl
from jax.experimental.pallas import tpu as pltpu
from jax.experimental.pallas import tpu_sc as plsc

N = 10000
E = 320000
D = 128
NC = 2
NS = 16
NW = NC * NS
CHUNK = 128
EDGES_PER_TILE = E // NS
FULL_CHUNKS = EDGES_PER_TILE // CHUNK
TAIL = EDGES_PER_TILE - FULL_CHUNKS * CHUNK
EDGES_PER_WORKER = E // NW
DEG_FULL = EDGES_PER_WORKER // CHUNK
DEG_TAIL = EDGES_PER_WORKER - DEG_FULL * CHUNK
ROWS_MAIN = 640
ROWS_LAST = N - (NS - 1) * ROWS_MAIN

_mesh = plsc.VectorSubcoreMesh(core_axis_name="c", subcore_axis_name="s")


def _per_tile_rows(s, fn):
    r0 = pl.multiple_of(s * ROWS_MAIN, 8)

    @pl.when(s < NS - 1)
    def _():
        fn(r0, ROWS_MAIN)

    @pl.when(s == NS - 1)
    def _():
        fn((NS - 1) * ROWS_MAIN, ROWS_LAST)


def _deg_body(dst_hbm, ones_hbm, zeros_hbm, out_hbm,
              didx, didx_t, ones_v, ones_t, acc):
    c = lax.axis_index("c")
    s = lax.axis_index("s")
    wid = c * NS + s

    _per_tile_rows(s, lambda r0, nr: pltpu.sync_copy(
        zeros_hbm.at[pl.ds(r0, nr)], acc.at[pl.ds(r0, nr)]))
    pltpu.sync_copy(ones_hbm, ones_v)
    pltpu.sync_copy(ones_hbm.at[pl.ds(0, DEG_TAIL)], ones_t)
    plsc.subcore_barrier()

    base = wid * EDGES_PER_WORKER

    def step(g, _):
        e0 = base + g * CHUNK
        pltpu.sync_copy(dst_hbm.at[pl.ds(e0, CHUNK)], didx)
        pltpu.sync_copy(ones_v, acc.at[didx], add=True)
        return 0
    lax.fori_loop(0, DEG_FULL, step, 0)
    if DEG_TAIL:
        e0 = base + DEG_FULL * CHUNK
        pltpu.sync_copy(dst_hbm.at[pl.ds(e0, DEG_TAIL)], didx_t)
        pltpu.sync_copy(ones_t, acc.at[didx_t], add=True)
    plsc.subcore_barrier()

    _per_tile_rows(s, lambda r0, nr: pltpu.sync_copy(
        acc.at[pl.ds(r0, nr)], out_hbm.at[c].at[pl.ds(r0, nr)]))


_deg_kernel = functools.partial(
    pl.kernel,
    out_type=jax.ShapeDtypeStruct((NC, N, D), jnp.float32),
    mesh=_mesh,
    scratch_types=[
        pltpu.VMEM((CHUNK,), jnp.int32),
        pltpu.VMEM((DEG_TAIL,), jnp.int32),
        pltpu.VMEM((CHUNK, D), jnp.float32),
        pltpu.VMEM((DEG_TAIL, D), jnp.float32),
        pltpu.VMEM_SHARED((N, D), jnp.float32),
    ],
)(_deg_body)


def _agg_body(feats_hbm, src_hbm, dst_hbm, zeros_hbm, out_hbm,
              sidx, didx, rows, sidx_t, didx_t, rows_t, acc, sem):
    c = lax.axis_index("c")
    s = lax.axis_index("s")

    _per_tile_rows(s, lambda r0, nr: pltpu.sync_copy(
        zeros_hbm.at[pl.ds(r0, nr)], acc.at[pl.ds(r0, nr)]))
    plsc.subcore_barrier()

    table = feats_hbm.at[c]
    base = s * EDGES_PER_TILE

    def step(g, _):
        e0 = base + g * CHUNK
        pltpu.sync_copy(src_hbm.at[pl.ds(e0, CHUNK)], sidx)
        pltpu.sync_copy(dst_hbm.at[pl.ds(e0, CHUNK)], didx)
        pltpu.async_copy(table.at[sidx], rows, sem).wait()
        pltpu.sync_copy(rows, acc.at[didx], add=True)
        return 0
    lax.fori_loop(0, FULL_CHUNKS, step, 0)
    if TAIL:
        e0 = base + FULL_CHUNKS * CHUNK
        pltpu.sync_copy(src_hbm.at[pl.ds(e0, TAIL)], sidx_t)
        pltpu.sync_copy(dst_hbm.at[pl.ds(e0, TAIL)], didx_t)
        pltpu.async_copy(table.at[sidx_t], rows_t, sem).wait()
        pltpu.sync_copy(rows_t, acc.at[didx_t], add=True)
    plsc.subcore_barrier()

    _per_tile_rows(s, lambda r0, nr: pltpu.sync_copy(
        acc.at[pl.ds(r0, nr)], out_hbm.at[c].at[pl.ds(r0, nr)]))


_agg_kernel = functools.partial(
    pl.kernel,
    out_type=jax.ShapeDtypeStruct((NC, N, D), jnp.float32),
    mesh=_mesh,
    scratch_types=[
        pltpu.VMEM((CHUNK,), jnp.int32),
        pltpu.VMEM((CHUNK,), jnp.int32),
        pltpu.VMEM((CHUNK, D), jnp.float32),
        pltpu.VMEM((TAIL,), jnp.int32),
        pltpu.VMEM((TAIL,), jnp.int32),
        pltpu.VMEM((TAIL, D), jnp.float32),
        pltpu.VMEM_SHARED((N, D), jnp.float32),
        pltpu.SemaphoreType.DMA,
    ],
)(_agg_body)


BLK = 1000


def _pre_body(degp_ref, x_ref, hx_ref, dis_ref, xnhn_ref):
    deg = degp_ref[0, :, 0:1] + degp_ref[1, :, 0:1]
    dis = lax.rsqrt(jnp.maximum(deg, 1.0))
    dis_ref[...] = dis
    xnhn_ref[0] = x_ref[...] * dis
    xnhn_ref[1] = hx_ref[...] * dis


def _tc_pre(degp, x, hx):
    return pl.pallas_call(
        _pre_body,
        grid=(N // BLK,),
        in_specs=[
            pl.BlockSpec((NC, BLK, D), lambda i: (0, i, 0)),
            pl.BlockSpec((BLK, D), lambda i: (i, 0)),
            pl.BlockSpec((BLK, D), lambda i: (i, 0)),
        ],
        out_specs=[
            pl.BlockSpec((BLK, 1), lambda i: (i, 0)),
            pl.BlockSpec((NC, BLK, D), lambda i: (0, i, 0)),
        ],
        out_shape=[
            jax.ShapeDtypeStruct((N, 1), jnp.float32),
            jax.ShapeDtypeStruct((NC, N, D), jnp.float32),
        ],
    )(degp, x, hx)


def _post_body(agg_ref, dis_ref, x_ref, hx_ref,
               w0x_ref, w1x_ref, w0h_ref, w1h_ref, bx_ref, bh_ref, out_ref):
    dis = dis_ref[...]
    tx1x = agg_ref[0] * (-dis)
    tx1h = agg_ref[1] * (-dis)
    x = x_ref[...]
    hx = hx_ref[...]
    f32 = jnp.float32
    gx = (jnp.dot(x, w0x_ref[...], preferred_element_type=f32)
          + jnp.dot(tx1x, w1x_ref[...], preferred_element_type=f32)
          + bx_ref[...])
    gh = (jnp.dot(hx, w0h_ref[...], preferred_element_type=f32)
          + jnp.dot(tx1h, w1h_ref[...], preferred_element_type=f32)
          + bh_ref[...])
    r = jax.nn.sigmoid(gx[:, 0:D] + gh[:, 0:D])
    u = jax.nn.sigmoid(gx[:, D:2 * D] + gh[:, D:2 * D])
    cg = jax.nn.sigmoid(gx[:, 2 * D:3 * D] + gh[:, 2 * D:3 * D] * r)
    out_ref[...] = u * hx + (1.0 - u) * cg


def _tc_post(agg, dis, x, hx, w0x, w1x, w0h, w1h, bx, bh):
    wspec = pl.BlockSpec((D, 3 * D), lambda i: (0, 0))
    bspec = pl.BlockSpec((1, 3 * D), lambda i: (0, 0))
    return pl.pallas_call(
        _post_body,
        grid=(N // BLK,),
        in_specs=[
            pl.BlockSpec((NC, BLK, D), lambda i: (0, i, 0)),
            pl.BlockSpec((BLK, 1), lambda i: (i, 0)),
            pl.BlockSpec((BLK, D), lambda i: (i, 0)),
            pl.BlockSpec((BLK, D), lambda i: (i, 0)),
            wspec, wspec, wspec, wspec, bspec, bspec,
        ],
        out_specs=pl.BlockSpec((BLK, D), lambda i: (i, 0)),
        out_shape=jax.ShapeDtypeStruct((N, D), jnp.float32),
    )(agg, dis, x, hx, w0x, w1x, w0h, w1h, bx, bh)


@jax.jit
def kernel(x, hx, edge_index,
           W0_r_x, W1_r_x, b_r_x, W0_r_h, W1_r_h, b_r_h,
           W0_u_x, W1_u_x, b_u_x, W0_u_h, W1_u_h, b_u_h,
           W0_c_x, W1_c_x, b_c_x, W0_c_h, W1_c_h, b_c_h):
    src = edge_index[0]
    dst = edge_index[1]

    w0x = jnp.concatenate([W0_r_x, W0_u_x, W0_c_x], axis=1)
    w1x = jnp.concatenate([W1_r_x, W1_u_x, W1_c_x], axis=1)
    w0h = jnp.concatenate([W0_r_h, W0_u_h, W0_c_h], axis=1)
    w1h = jnp.concatenate([W1_r_h, W1_u_h, W1_c_h], axis=1)
    bx = jnp.concatenate([b_r_x, b_u_x, b_c_x])[None, :]
    bh = jnp.concatenate([b_r_h, b_u_h, b_c_h])[None, :]

    ones_feat = jnp.ones((CHUNK, D), jnp.float32)
    zeros_feat = jnp.zeros((N, D), jnp.float32)

    degp = _deg_kernel(dst, ones_feat, zeros_feat)
    dis, xnhn = _tc_pre(degp, x, hx)
    agg = _agg_kernel(xnhn, src, dst, zeros_feat)
    return _tc_post(agg, dis, x, hx, w0x, w1x, w0h, w1h, bx, bh)


# speedup vs baseline: 5.5692x; 1.4180x over previous
import functools

import jax
import jax.numpy as jnp
from jax import lax
from jax.experimental import pallas as pl
from jax.experimental.pallas import tpu as pltpu
from jax.experimental.pallas import tpu_sc as plsc

N = 10000
E = 320000
D = 128
NC = 2
NS = 16
NW = NC * NS
CHUNK = 128
EDGES_PER_TILE = E // NS
FULL_CHUNKS = EDGES_PER_TILE // CHUNK
TAIL = EDGES_PER_TILE - FULL_CHUNKS * CHUNK
EDGES_PER_WORKER = E // NW
DEG_FULL = EDGES_PER_WORKER // CHUNK
DEG_TAIL = EDGES_PER_WORKER - DEG_FULL * CHUNK
ROWS_MAIN = 640
ROWS_LAST = N - (NS - 1) * ROWS_MAIN

_mesh = plsc.VectorSubcoreMesh(core_axis_name="c", subcore_axis_name="s")


def _per_tile_rows(s, fn):
    r0 = pl.multiple_of(s * ROWS_MAIN, 8)

    @pl.when(s < NS - 1)
    def _():
        fn(r0, ROWS_MAIN)

    @pl.when(s == NS - 1)
    def _():
        fn((NS - 1) * ROWS_MAIN, ROWS_LAST)


def _deg_body(dst_hbm, ones_hbm, zeros_hbm, out_hbm,
              didx, didx_t, ones_v, ones_t, acc):
    c = lax.axis_index("c")
    s = lax.axis_index("s")
    wid = c * NS + s

    _per_tile_rows(s, lambda r0, nr: pltpu.sync_copy(
        zeros_hbm.at[pl.ds(r0, nr)], acc.at[pl.ds(r0, nr)]))
    pltpu.sync_copy(ones_hbm, ones_v)
    pltpu.sync_copy(ones_hbm.at[pl.ds(0, DEG_TAIL)], ones_t)
    plsc.subcore_barrier()

    base = wid * EDGES_PER_WORKER

    def step(g, _):
        e0 = base + g * CHUNK
        pltpu.sync_copy(dst_hbm.at[pl.ds(e0, CHUNK)], didx)
        pltpu.sync_copy(ones_v, acc.at[didx], add=True)
        return 0
    lax.fori_loop(0, DEG_FULL, step, 0)
    if DEG_TAIL:
        e0 = base + DEG_FULL * CHUNK
        pltpu.sync_copy(dst_hbm.at[pl.ds(e0, DEG_TAIL)], didx_t)
        pltpu.sync_copy(ones_t, acc.at[didx_t], add=True)
    plsc.subcore_barrier()

    _per_tile_rows(s, lambda r0, nr: pltpu.sync_copy(
        acc.at[pl.ds(r0, nr)], out_hbm.at[c].at[pl.ds(r0, nr)]))


_deg_kernel = functools.partial(
    pl.kernel,
    out_type=jax.ShapeDtypeStruct((NC, N, D), jnp.float32),
    mesh=_mesh,
    scratch_types=[
        pltpu.VMEM((CHUNK,), jnp.int32),
        pltpu.VMEM((DEG_TAIL,), jnp.int32),
        pltpu.VMEM((CHUNK, D), jnp.float32),
        pltpu.VMEM((DEG_TAIL, D), jnp.float32),
        pltpu.VMEM_SHARED((N, D), jnp.float32),
    ],
)(_deg_body)


NBUF = 2


def _agg_body(feats_hbm, src_hbm, dst_hbm, zeros_hbm, out_hbm, *scr):
    sidxs = scr[0:NBUF]
    didxs = scr[NBUF:2 * NBUF]
    rowss = scr[2 * NBUF:3 * NBUF]
    sidx_t, didx_t, rows_t, acc = scr[3 * NBUF:3 * NBUF + 4]
    sems = scr[3 * NBUF + 4:]

    c = lax.axis_index("c")
    s = lax.axis_index("s")

    _per_tile_rows(s, lambda r0, nr: pltpu.sync_copy(
        zeros_hbm.at[pl.ds(r0, nr)], acc.at[pl.ds(r0, nr)]))
    plsc.subcore_barrier()

    table = feats_hbm.at[c]
    base = s * EDGES_PER_TILE

    def fetch(g, b):
        e0 = base + g * CHUNK
        pltpu.sync_copy(src_hbm.at[pl.ds(e0, CHUNK)], sidxs[b])
        pltpu.sync_copy(dst_hbm.at[pl.ds(e0, CHUNK)], didxs[b])
        pltpu.async_copy(table.at[sidxs[b]], rowss[b], sems[b])

    for b in range(NBUF):
        fetch(b, b)

    def outer(g0, _):
        for b in range(NBUF):
            g = g0 * NBUF + b
            pltpu.make_async_copy(table.at[sidxs[b]], rowss[b],
                                  sems[b]).wait()
            pltpu.sync_copy(rowss[b], acc.at[didxs[b]], add=True)

            @pl.when(g + NBUF < FULL_CHUNKS)
            def _():
                fetch(g + NBUF, b)
        return 0
    lax.fori_loop(0, FULL_CHUNKS // NBUF, outer, 0)

    if TAIL:
        e0 = base + FULL_CHUNKS * CHUNK
        pltpu.sync_copy(src_hbm.at[pl.ds(e0, TAIL)], sidx_t)
        pltpu.sync_copy(dst_hbm.at[pl.ds(e0, TAIL)], didx_t)
        pltpu.async_copy(table.at[sidx_t], rows_t, sems[0]).wait()
        pltpu.sync_copy(rows_t, acc.at[didx_t], add=True)
    plsc.subcore_barrier()

    _per_tile_rows(s, lambda r0, nr: pltpu.sync_copy(
        acc.at[pl.ds(r0, nr)], out_hbm.at[c].at[pl.ds(r0, nr)]))


_agg_kernel = functools.partial(
    pl.kernel,
    out_type=jax.ShapeDtypeStruct((NC, N, D), jnp.float32),
    mesh=_mesh,
    scratch_types=(
        [pltpu.VMEM((CHUNK,), jnp.int32) for _ in range(2 * NBUF)]
        + [pltpu.VMEM((CHUNK, D), jnp.float32) for _ in range(NBUF)]
        + [
            pltpu.VMEM((TAIL,), jnp.int32),
            pltpu.VMEM((TAIL,), jnp.int32),
            pltpu.VMEM((TAIL, D), jnp.float32),
            pltpu.VMEM_SHARED((N, D), jnp.float32),
        ]
        + [pltpu.SemaphoreType.DMA for _ in range(NBUF)]
    ),
)(_agg_body)


BLK = 1000


def _pre_body(degp_ref, x_ref, hx_ref, dis_ref, xnhn_ref):
    deg = degp_ref[0, :, 0:1] + degp_ref[1, :, 0:1]
    dis = lax.rsqrt(jnp.maximum(deg, 1.0))
    dis_ref[...] = dis
    xnhn_ref[0] = x_ref[...] * dis
    xnhn_ref[1] = hx_ref[...] * dis


def _tc_pre(degp, x, hx):
    return pl.pallas_call(
        _pre_body,
        grid=(N // BLK,),
        in_specs=[
            pl.BlockSpec((NC, BLK, D), lambda i: (0, i, 0)),
            pl.BlockSpec((BLK, D), lambda i: (i, 0)),
            pl.BlockSpec((BLK, D), lambda i: (i, 0)),
        ],
        out_specs=[
            pl.BlockSpec((BLK, 1), lambda i: (i, 0)),
            pl.BlockSpec((NC, BLK, D), lambda i: (0, i, 0)),
        ],
        out_shape=[
            jax.ShapeDtypeStruct((N, 1), jnp.float32),
            jax.ShapeDtypeStruct((NC, N, D), jnp.float32),
        ],
    )(degp, x, hx)


def _post_body(agg_ref, dis_ref, x_ref, hx_ref,
               w0x_ref, w1x_ref, w0h_ref, w1h_ref, bx_ref, bh_ref, out_ref):
    dis = dis_ref[...]
    tx1x = agg_ref[0] * (-dis)
    tx1h = agg_ref[1] * (-dis)
    x = x_ref[...]
    hx = hx_ref[...]
    f32 = jnp.float32
    gx = (jnp.dot(x, w0x_ref[...], preferred_element_type=f32)
          + jnp.dot(tx1x, w1x_ref[...], preferred_element_type=f32)
          + bx_ref[...])
    gh = (jnp.dot(hx, w0h_ref[...], preferred_element_type=f32)
          + jnp.dot(tx1h, w1h_ref[...], preferred_element_type=f32)
          + bh_ref[...])
    r = jax.nn.sigmoid(gx[:, 0:D] + gh[:, 0:D])
    u = jax.nn.sigmoid(gx[:, D:2 * D] + gh[:, D:2 * D])
    cg = jax.nn.sigmoid(gx[:, 2 * D:3 * D] + gh[:, 2 * D:3 * D] * r)
    out_ref[...] = u * hx + (1.0 - u) * cg


def _tc_post(agg, dis, x, hx, w0x, w1x, w0h, w1h, bx, bh):
    wspec = pl.BlockSpec((D, 3 * D), lambda i: (0, 0))
    bspec = pl.BlockSpec((1, 3 * D), lambda i: (0, 0))
    return pl.pallas_call(
        _post_body,
        grid=(N // BLK,),
        in_specs=[
            pl.BlockSpec((NC, BLK, D), lambda i: (0, i, 0)),
            pl.BlockSpec((BLK, 1), lambda i: (i, 0)),
            pl.BlockSpec((BLK, D), lambda i: (i, 0)),
            pl.BlockSpec((BLK, D), lambda i: (i, 0)),
            wspec, wspec, wspec, wspec, bspec, bspec,
        ],
        out_specs=pl.BlockSpec((BLK, D), lambda i: (i, 0)),
        out_shape=jax.ShapeDtypeStruct((N, D), jnp.float32),
    )(agg, dis, x, hx, w0x, w1x, w0h, w1h, bx, bh)


@jax.jit
def kernel(x, hx, edge_index,
           W0_r_x, W1_r_x, b_r_x, W0_r_h, W1_r_h, b_r_h,
           W0_u_x, W1_u_x, b_u_x, W0_u_h, W1_u_h, b_u_h,
           W0_c_x, W1_c_x, b_c_x, W0_c_h, W1_c_h, b_c_h):
    src = edge_index[0]
    dst = edge_index[1]

    w0x = jnp.concatenate([W0_r_x, W0_u_x, W0_c_x], axis=1)
    w1x = jnp.concatenate([W1_r_x, W1_u_x, W1_c_x], axis=1)
    w0h = jnp.concatenate([W0_r_h, W0_u_h, W0_c_h], axis=1)
    w1h = jnp.concatenate([W1_r_h, W1_u_h, W1_c_h], axis=1)
    bx = jnp.concatenate([b_r_x, b_u_x, b_c_x])[None, :]
    bh = jnp.concatenate([b_r_h, b_u_h, b_c_h])[None, :]

    ones_feat = jnp.ones((CHUNK, D), jnp.float32)
    zeros_feat = jnp.zeros((N, D), jnp.float32)

    degp = _deg_kernel(dst, ones_feat, zeros_feat)
    dis, xnhn = _tc_pre(degp, x, hx)
    agg = _agg_kernel(xnhn, src, dst, zeros_feat)
    return _tc_post(agg, dis, x, hx, w0x, w1x, w0h, w1h, bx, bh)


# speedup vs baseline: 6.8032x; 1.2216x over previous
import functools

import jax
import jax.numpy as jnp
from jax import lax
from jax.experimental import pallas as pl
from jax.experimental.pallas import tpu as pltpu
from jax.experimental.pallas import tpu_sc as plsc

N = 10000
E = 320000
D = 128
NC = 2
NS = 16
NW = NC * NS
CHUNK = 128
EDGES_PER_TILE = E // NS
FULL_CHUNKS = EDGES_PER_TILE // CHUNK
TAIL = EDGES_PER_TILE - FULL_CHUNKS * CHUNK
EDGES_PER_WORKER = E // NW
DEG_FULL = EDGES_PER_WORKER // CHUNK
DEG_TAIL = EDGES_PER_WORKER - DEG_FULL * CHUNK
IDX_CHUNKS = 13
IDX_BLK = IDX_CHUNKS * CHUNK
AGG_BLKS = FULL_CHUNKS // IDX_CHUNKS
DEG_BLKS = DEG_FULL // IDX_CHUNKS
ROWS_MAIN = 640
ROWS_LAST = N - (NS - 1) * ROWS_MAIN

_mesh = plsc.VectorSubcoreMesh(core_axis_name="c", subcore_axis_name="s")


def _per_tile_rows(s, fn):
    r0 = pl.multiple_of(s * ROWS_MAIN, 8)

    @pl.when(s < NS - 1)
    def _():
        fn(r0, ROWS_MAIN)

    @pl.when(s == NS - 1)
    def _():
        fn((NS - 1) * ROWS_MAIN, ROWS_LAST)


def _deg_body(dst_hbm, ones_hbm, zeros_hbm, out_hbm,
              didx, didx_t, ones_v, ones_t, acc):
    c = lax.axis_index("c")
    s = lax.axis_index("s")
    wid = c * NS + s

    _per_tile_rows(s, lambda r0, nr: pltpu.sync_copy(
        zeros_hbm.at[pl.ds(r0, nr)], acc.at[pl.ds(r0, nr)]))
    pltpu.sync_copy(ones_hbm, ones_v)
    pltpu.sync_copy(ones_hbm.at[pl.ds(0, DEG_TAIL)], ones_t)
    plsc.subcore_barrier()

    base = wid * EDGES_PER_WORKER

    def blk(b, _):
        e0 = base + b * IDX_BLK
        pltpu.sync_copy(dst_hbm.at[pl.ds(e0, IDX_BLK)], didx)
        for k in range(IDX_CHUNKS):
            pltpu.sync_copy(ones_v, acc.at[didx.at[pl.ds(k * CHUNK, CHUNK)]],
                            add=True)
        return 0
    lax.fori_loop(0, DEG_BLKS, blk, 0)
    if DEG_TAIL:
        e0 = base + DEG_FULL * CHUNK
        pltpu.sync_copy(dst_hbm.at[pl.ds(e0, DEG_TAIL)], didx_t)
        pltpu.sync_copy(ones_t, acc.at[didx_t], add=True)
    plsc.subcore_barrier()

    _per_tile_rows(s, lambda r0, nr: pltpu.sync_copy(
        acc.at[pl.ds(r0, nr)], out_hbm.at[c].at[pl.ds(r0, nr)]))


_deg_kernel = functools.partial(
    pl.kernel,
    out_type=jax.ShapeDtypeStruct((NC, N, D), jnp.float32),
    mesh=_mesh,
    scratch_types=[
        pltpu.VMEM((IDX_BLK,), jnp.int32),
        pltpu.VMEM((DEG_TAIL,), jnp.int32),
        pltpu.VMEM((CHUNK, D), jnp.float32),
        pltpu.VMEM((DEG_TAIL, D), jnp.float32),
        pltpu.VMEM_SHARED((N, D), jnp.float32),
    ],
)(_deg_body)


NBUF = 2


def _agg_body(feats_hbm, src_hbm, dst_hbm, zeros_hbm, out_hbm, *scr):
    sidx, didx = scr[0], scr[1]
    rowss = scr[2:2 + NBUF]
    sidx_t, didx_t, rows_t, acc = scr[2 + NBUF:6 + NBUF]
    sems = scr[6 + NBUF:]

    c = lax.axis_index("c")
    s = lax.axis_index("s")

    _per_tile_rows(s, lambda r0, nr: pltpu.sync_copy(
        zeros_hbm.at[pl.ds(r0, nr)], acc.at[pl.ds(r0, nr)]))
    plsc.subcore_barrier()

    table = feats_hbm.at[c]
    base = s * EDGES_PER_TILE

    def sl(ref, k):
        return ref.at[pl.ds(k * CHUNK, CHUNK)]

    def blk(b, _):
        e0 = base + b * IDX_BLK
        pltpu.sync_copy(src_hbm.at[pl.ds(e0, IDX_BLK)], sidx)
        pltpu.sync_copy(dst_hbm.at[pl.ds(e0, IDX_BLK)], didx)
        for r in range(NBUF):
            pltpu.async_copy(table.at[sl(sidx, r)], rowss[r], sems[r])
        for k in range(IDX_CHUNKS):
            r = k % NBUF
            pltpu.make_async_copy(table.at[sl(sidx, k)], rowss[r],
                                  sems[r]).wait()
            pltpu.sync_copy(rowss[r], acc.at[sl(didx, k)], add=True)
            if k + NBUF < IDX_CHUNKS:
                pltpu.async_copy(table.at[sl(sidx, k + NBUF)], rowss[r],
                                 sems[r])
        return 0
    lax.fori_loop(0, AGG_BLKS, blk, 0)

    if TAIL:
        e0 = base + FULL_CHUNKS * CHUNK
        pltpu.sync_copy(src_hbm.at[pl.ds(e0, TAIL)], sidx_t)
        pltpu.sync_copy(dst_hbm.at[pl.ds(e0, TAIL)], didx_t)
        pltpu.async_copy(table.at[sidx_t], rows_t, sems[0]).wait()
        pltpu.sync_copy(rows_t, acc.at[didx_t], add=True)
    plsc.subcore_barrier()

    _per_tile_rows(s, lambda r0, nr: pltpu.sync_copy(
        acc.at[pl.ds(r0, nr)], out_hbm.at[c].at[pl.ds(r0, nr)]))


_agg_kernel = functools.partial(
    pl.kernel,
    out_type=jax.ShapeDtypeStruct((NC, N, D), jnp.float32),
    mesh=_mesh,
    scratch_types=(
        [pltpu.VMEM((IDX_BLK,), jnp.int32) for _ in range(2)]
        + [pltpu.VMEM((CHUNK, D), jnp.float32) for _ in range(NBUF)]
        + [
            pltpu.VMEM((TAIL,), jnp.int32),
            pltpu.VMEM((TAIL,), jnp.int32),
            pltpu.VMEM((TAIL, D), jnp.float32),
            pltpu.VMEM_SHARED((N, D), jnp.float32),
        ]
        + [pltpu.SemaphoreType.DMA for _ in range(NBUF)]
    ),
)(_agg_body)


BLK = 1000


def _pre_body(degp_ref, x_ref, hx_ref, dis_ref, xnhn_ref):
    deg = degp_ref[0, :, 0:1] + degp_ref[1, :, 0:1]
    dis = lax.rsqrt(jnp.maximum(deg, 1.0))
    dis_ref[...] = dis
    xnhn_ref[0] = x_ref[...] * dis
    xnhn_ref[1] = hx_ref[...] * dis


def _tc_pre(degp, x, hx):
    return pl.pallas_call(
        _pre_body,
        grid=(N // BLK,),
        in_specs=[
            pl.BlockSpec((NC, BLK, D), lambda i: (0, i, 0)),
            pl.BlockSpec((BLK, D), lambda i: (i, 0)),
            pl.BlockSpec((BLK, D), lambda i: (i, 0)),
        ],
        out_specs=[
            pl.BlockSpec((BLK, 1), lambda i: (i, 0)),
            pl.BlockSpec((NC, BLK, D), lambda i: (0, i, 0)),
        ],
        out_shape=[
            jax.ShapeDtypeStruct((N, 1), jnp.float32),
            jax.ShapeDtypeStruct((NC, N, D), jnp.float32),
        ],
    )(degp, x, hx)


def _post_body(agg_ref, dis_ref, x_ref, hx_ref,
               w0x_ref, w1x_ref, w0h_ref, w1h_ref, bx_ref, bh_ref, out_ref):
    dis = dis_ref[...]
    tx1x = agg_ref[0] * (-dis)
    tx1h = agg_ref[1] * (-dis)
    x = x_ref[...]
    hx = hx_ref[...]
    f32 = jnp.float32
    gx = (jnp.dot(x, w0x_ref[...], preferred_element_type=f32)
          + jnp.dot(tx1x, w1x_ref[...], preferred_element_type=f32)
          + bx_ref[...])
    gh = (jnp.dot(hx, w0h_ref[...], preferred_element_type=f32)
          + jnp.dot(tx1h, w1h_ref[...], preferred_element_type=f32)
          + bh_ref[...])
    r = jax.nn.sigmoid(gx[:, 0:D] + gh[:, 0:D])
    u = jax.nn.sigmoid(gx[:, D:2 * D] + gh[:, D:2 * D])
    cg = jax.nn.sigmoid(gx[:, 2 * D:3 * D] + gh[:, 2 * D:3 * D] * r)
    out_ref[...] = u * hx + (1.0 - u) * cg


def _tc_post(agg, dis, x, hx, w0x, w1x, w0h, w1h, bx, bh):
    wspec = pl.BlockSpec((D, 3 * D), lambda i: (0, 0))
    bspec = pl.BlockSpec((1, 3 * D), lambda i: (0, 0))
    return pl.pallas_call(
        _post_body,
        grid=(N // BLK,),
        in_specs=[
            pl.BlockSpec((NC, BLK, D), lambda i: (0, i, 0)),
            pl.BlockSpec((BLK, 1), lambda i: (i, 0)),
            pl.BlockSpec((BLK, D), lambda i: (i, 0)),
            pl.BlockSpec((BLK, D), lambda i: (i, 0)),
            wspec, wspec, wspec, wspec, bspec, bspec,
        ],
        out_specs=pl.BlockSpec((BLK, D), lambda i: (i, 0)),
        out_shape=jax.ShapeDtypeStruct((N, D), jnp.float32),
    )(agg, dis, x, hx, w0x, w1x, w0h, w1h, bx, bh)


@jax.jit
def kernel(x, hx, edge_index,
           W0_r_x, W1_r_x, b_r_x, W0_r_h, W1_r_h, b_r_h,
           W0_u_x, W1_u_x, b_u_x, W0_u_h, W1_u_h, b_u_h,
           W0_c_x, W1_c_x, b_c_x, W0_c_h, W1_c_h, b_c_h):
    src = edge_index[0]
    dst = edge_index[1]

    w0x = jnp.concatenate([W0_r_x, W0_u_x, W0_c_x], axis=1)
    w1x = jnp.concatenate([W1_r_x, W1_u_x, W1_c_x], axis=1)
    w0h = jnp.concatenate([W0_r_h, W0_u_h, W0_c_h], axis=1)
    w1h = jnp.concatenate([W1_r_h, W1_u_h, W1_c_h], axis=1)
    bx = jnp.concatenate([b_r_x, b_u_x, b_c_x])[None, :]
    bh = jnp.concatenate([b_r_h, b_u_h, b_c_h])[None, :]

    ones_feat = jnp.ones((CHUNK, D), jnp.float32)
    zeros_feat = jnp.zeros((N, D), jnp.float32)

    degp = _deg_kernel(dst, ones_feat, zeros_feat)
    dis, xnhn = _tc_pre(degp, x, hx)
    agg = _agg_kernel(xnhn, src, dst, zeros_feat)
    return _tc_post(agg, dis, x, hx, w0x, w1x, w0h, w1h, bx, bh)


# speedup vs baseline: 7.2285x; 1.0625x over previous
import functools

import jax
import jax.numpy as jnp
from jax import lax
from jax.experimental import pallas as pl
from jax.experimental.pallas import tpu as pltpu
from jax.experimental.pallas import tpu_sc as plsc

N = 10000
E = 320000
D = 128
NC = 2
NS = 16
NW = NC * NS
CHUNK = 128
EDGES_PER_TILE = E // NS
FULL_CHUNKS = EDGES_PER_TILE // CHUNK
TAIL = EDGES_PER_TILE - FULL_CHUNKS * CHUNK
EDGES_PER_WORKER = E // NW
DEG_FULL = EDGES_PER_WORKER // CHUNK
DEG_TAIL = EDGES_PER_WORKER - DEG_FULL * CHUNK
IDX_CHUNKS = 13
IDX_BLK = IDX_CHUNKS * CHUNK
DEG_BLKS = DEG_FULL // IDX_CHUNKS
AGG_IDXC = 12
AGG_IDXB = AGG_IDXC * CHUNK
AGG_BLKS = FULL_CHUNKS // AGG_IDXC
ROWS_MAIN = 640
ROWS_LAST = N - (NS - 1) * ROWS_MAIN

_mesh = plsc.VectorSubcoreMesh(core_axis_name="c", subcore_axis_name="s")


def _per_tile_rows(s, fn):
    r0 = pl.multiple_of(s * ROWS_MAIN, 8)

    @pl.when(s < NS - 1)
    def _():
        fn(r0, ROWS_MAIN)

    @pl.when(s == NS - 1)
    def _():
        fn((NS - 1) * ROWS_MAIN, ROWS_LAST)


def _deg_body(dst_hbm, ones_hbm, zeros_hbm, out_hbm,
              didx, didx_t, ones_v, ones_t, acc):
    c = lax.axis_index("c")
    s = lax.axis_index("s")
    wid = c * NS + s

    _per_tile_rows(s, lambda r0, nr: pltpu.sync_copy(
        zeros_hbm.at[pl.ds(r0, nr)], acc.at[pl.ds(r0, nr)]))
    pltpu.sync_copy(ones_hbm, ones_v)
    pltpu.sync_copy(ones_hbm.at[pl.ds(0, DEG_TAIL)], ones_t)
    plsc.subcore_barrier()

    base = wid * EDGES_PER_WORKER

    def blk(b, _):
        e0 = base + b * IDX_BLK
        pltpu.sync_copy(dst_hbm.at[pl.ds(e0, IDX_BLK)], didx)
        for k in range(IDX_CHUNKS):
            pltpu.sync_copy(ones_v, acc.at[didx.at[pl.ds(k * CHUNK, CHUNK)]],
                            add=True)
        return 0
    lax.fori_loop(0, DEG_BLKS, blk, 0)
    if DEG_TAIL:
        e0 = base + DEG_FULL * CHUNK
        pltpu.sync_copy(dst_hbm.at[pl.ds(e0, DEG_TAIL)], didx_t)
        pltpu.sync_copy(ones_t, acc.at[didx_t], add=True)
    plsc.subcore_barrier()

    _per_tile_rows(s, lambda r0, nr: pltpu.sync_copy(
        acc.at[pl.ds(r0, nr)], out_hbm.at[c].at[pl.ds(r0, nr)]))


_deg_kernel = functools.partial(
    pl.kernel,
    out_type=jax.ShapeDtypeStruct((NC, N, D), jnp.float32),
    mesh=_mesh,
    scratch_types=[
        pltpu.VMEM((IDX_BLK,), jnp.int32),
        pltpu.VMEM((DEG_TAIL,), jnp.int32),
        pltpu.VMEM((CHUNK, D), jnp.float32),
        pltpu.VMEM((DEG_TAIL, D), jnp.float32),
        pltpu.VMEM_SHARED((N, D), jnp.float32),
    ],
)(_deg_body)


NBUF = 2


def _agg_body(feats_hbm, src_hbm, dst_hbm, zeros_hbm, out_hbm, *scr):
    sidx, didx = scr[0], scr[1]
    rowss = scr[2:2 + NBUF]
    sidx_t, didx_t, rows_t, acc = scr[2 + NBUF:6 + NBUF]
    sems = scr[6 + NBUF:]

    c = lax.axis_index("c")
    s = lax.axis_index("s")

    _per_tile_rows(s, lambda r0, nr: pltpu.sync_copy(
        zeros_hbm.at[pl.ds(r0, nr)], acc.at[pl.ds(r0, nr)]))
    plsc.subcore_barrier()

    table = feats_hbm.at[c]
    base = s * EDGES_PER_TILE

    def idx_load(b, off):
        e0 = base + b * AGG_IDXB
        pltpu.sync_copy(src_hbm.at[pl.ds(e0, AGG_IDXB)],
                        sidx.at[pl.ds(off, AGG_IDXB)])
        pltpu.sync_copy(dst_hbm.at[pl.ds(e0, AGG_IDXB)],
                        didx.at[pl.ds(off, AGG_IDXB)])

    def gat(off_elems, r):
        pltpu.async_copy(table.at[sidx.at[pl.ds(off_elems, CHUNK)]],
                         rowss[r], sems[r])

    idx_load(0, 0)
    for r in range(NBUF):
        gat(r * CHUNK, r)

    def blk(b, _):
        off_cur = (b % 2) * AGG_IDXB
        off_nxt = ((b + 1) % 2) * AGG_IDXB

        @pl.when(b + 1 < AGG_BLKS)
        def _():
            idx_load(b + 1, off_nxt)

        for k in range(AGG_IDXC):
            r = k % NBUF
            o = off_cur + k * CHUNK
            pltpu.make_async_copy(table.at[sidx.at[pl.ds(o, CHUNK)]],
                                  rowss[r], sems[r]).wait()
            pltpu.sync_copy(rowss[r], acc.at[didx.at[pl.ds(o, CHUNK)]],
                            add=True)
            kn = k + NBUF
            if kn < AGG_IDXC:
                gat(off_cur + kn * CHUNK, r)
            else:
                @pl.when(b + 1 < AGG_BLKS)
                def _():
                    gat(off_nxt + (kn - AGG_IDXC) * CHUNK, r)
        return 0
    lax.fori_loop(0, AGG_BLKS, blk, 0)

    if TAIL:
        e0 = base + FULL_CHUNKS * CHUNK
        pltpu.sync_copy(src_hbm.at[pl.ds(e0, TAIL)], sidx_t)
        pltpu.sync_copy(dst_hbm.at[pl.ds(e0, TAIL)], didx_t)
        pltpu.async_copy(table.at[sidx_t], rows_t, sems[0]).wait()
        pltpu.sync_copy(rows_t, acc.at[didx_t], add=True)
    plsc.subcore_barrier()

    _per_tile_rows(s, lambda r0, nr: pltpu.sync_copy(
        acc.at[pl.ds(r0, nr)], out_hbm.at[c].at[pl.ds(r0, nr)]))


_agg_kernel = functools.partial(
    pl.kernel,
    out_type=jax.ShapeDtypeStruct((NC, N, D), jnp.float32),
    mesh=_mesh,
    scratch_types=(
        [pltpu.VMEM((2 * AGG_IDXB,), jnp.int32) for _ in range(2)]
        + [pltpu.VMEM((CHUNK, D), jnp.float32) for _ in range(NBUF)]
        + [
            pltpu.VMEM((TAIL,), jnp.int32),
            pltpu.VMEM((TAIL,), jnp.int32),
            pltpu.VMEM((TAIL, D), jnp.float32),
            pltpu.VMEM_SHARED((N, D), jnp.float32),
        ]
        + [pltpu.SemaphoreType.DMA for _ in range(NBUF)]
    ),
)(_agg_body)


BLK = 1000


def _pre_body(degp_ref, x_ref, hx_ref, dis_ref, xnhn_ref):
    deg = degp_ref[0, :, 0:1] + degp_ref[1, :, 0:1]
    dis = lax.rsqrt(jnp.maximum(deg, 1.0))
    dis_ref[...] = dis
    xnhn_ref[0] = x_ref[...] * dis
    xnhn_ref[1] = hx_ref[...] * dis


def _tc_pre(degp, x, hx):
    return pl.pallas_call(
        _pre_body,
        grid=(N // BLK,),
        in_specs=[
            pl.BlockSpec((NC, BLK, D), lambda i: (0, i, 0)),
            pl.BlockSpec((BLK, D), lambda i: (i, 0)),
            pl.BlockSpec((BLK, D), lambda i: (i, 0)),
        ],
        out_specs=[
            pl.BlockSpec((BLK, 1), lambda i: (i, 0)),
            pl.BlockSpec((NC, BLK, D), lambda i: (0, i, 0)),
        ],
        out_shape=[
            jax.ShapeDtypeStruct((N, 1), jnp.float32),
            jax.ShapeDtypeStruct((NC, N, D), jnp.float32),
        ],
    )(degp, x, hx)


def _post_body(agg_ref, dis_ref, x_ref, hx_ref,
               w0x_ref, w1x_ref, w0h_ref, w1h_ref, bx_ref, bh_ref, out_ref):
    dis = dis_ref[...]
    tx1x = agg_ref[0] * (-dis)
    tx1h = agg_ref[1] * (-dis)
    x = x_ref[...]
    hx = hx_ref[...]
    f32 = jnp.float32
    gx = (jnp.dot(x, w0x_ref[...], preferred_element_type=f32)
          + jnp.dot(tx1x, w1x_ref[...], preferred_element_type=f32)
          + bx_ref[...])
    gh = (jnp.dot(hx, w0h_ref[...], preferred_element_type=f32)
          + jnp.dot(tx1h, w1h_ref[...], preferred_element_type=f32)
          + bh_ref[...])
    r = jax.nn.sigmoid(gx[:, 0:D] + gh[:, 0:D])
    u = jax.nn.sigmoid(gx[:, D:2 * D] + gh[:, D:2 * D])
    cg = jax.nn.sigmoid(gx[:, 2 * D:3 * D] + gh[:, 2 * D:3 * D] * r)
    out_ref[...] = u * hx + (1.0 - u) * cg


def _tc_post(agg, dis, x, hx, w0x, w1x, w0h, w1h, bx, bh):
    wspec = pl.BlockSpec((D, 3 * D), lambda i: (0, 0))
    bspec = pl.BlockSpec((1, 3 * D), lambda i: (0, 0))
    return pl.pallas_call(
        _post_body,
        grid=(N // BLK,),
        in_specs=[
            pl.BlockSpec((NC, BLK, D), lambda i: (0, i, 0)),
            pl.BlockSpec((BLK, 1), lambda i: (i, 0)),
            pl.BlockSpec((BLK, D), lambda i: (i, 0)),
            pl.BlockSpec((BLK, D), lambda i: (i, 0)),
            wspec, wspec, wspec, wspec, bspec, bspec,
        ],
        out_specs=pl.BlockSpec((BLK, D), lambda i: (i, 0)),
        out_shape=jax.ShapeDtypeStruct((N, D), jnp.float32),
    )(agg, dis, x, hx, w0x, w1x, w0h, w1h, bx, bh)


@jax.jit
def kernel(x, hx, edge_index,
           W0_r_x, W1_r_x, b_r_x, W0_r_h, W1_r_h, b_r_h,
           W0_u_x, W1_u_x, b_u_x, W0_u_h, W1_u_h, b_u_h,
           W0_c_x, W1_c_x, b_c_x, W0_c_h, W1_c_h, b_c_h):
    src = edge_index[0]
    dst = edge_index[1]

    w0x = jnp.concatenate([W0_r_x, W0_u_x, W0_c_x], axis=1)
    w1x = jnp.concatenate([W1_r_x, W1_u_x, W1_c_x], axis=1)
    w0h = jnp.concatenate([W0_r_h, W0_u_h, W0_c_h], axis=1)
    w1h = jnp.concatenate([W1_r_h, W1_u_h, W1_c_h], axis=1)
    bx = jnp.concatenate([b_r_x, b_u_x, b_c_x])[None, :]
    bh = jnp.concatenate([b_r_h, b_u_h, b_c_h])[None, :]

    ones_feat = jnp.ones((CHUNK, D), jnp.float32)
    zeros_feat = jnp.zeros((N, D), jnp.float32)

    degp = _deg_kernel(dst, ones_feat, zeros_feat)
    dis, xnhn = _tc_pre(degp, x, hx)
    agg = _agg_kernel(xnhn, src, dst, zeros_feat)
    return _tc_post(agg, dis, x, hx, w0x, w1x, w0h, w1h, bx, bh)


# speedup vs baseline: 7.4758x; 1.0342x over previous
import functools

import jax
import jax.numpy as jnp
from jax import lax
from jax.experimental import pallas as pl
from jax.experimental.pallas import tpu as pltpu
from jax.experimental.pallas import tpu_sc as plsc

N = 10000
E = 320000
D = 128
NC = 2
NS = 16
NW = NC * NS
CHUNK = 128
EDGES_PER_TILE = E // NS
FULL_CHUNKS = EDGES_PER_TILE // CHUNK
TAIL = EDGES_PER_TILE - FULL_CHUNKS * CHUNK
EDGES_PER_WORKER = E // NW
DEG_FULL = EDGES_PER_WORKER // CHUNK
DEG_TAIL = EDGES_PER_WORKER - DEG_FULL * CHUNK
IDX_CHUNKS = 13
IDX_BLK = IDX_CHUNKS * CHUNK
DEG_BLKS = DEG_FULL // IDX_CHUNKS
AGG_IDXC = 12
AGG_IDXB = AGG_IDXC * CHUNK
AGG_BLKS = FULL_CHUNKS // AGG_IDXC
ROWS_MAIN = 640
ROWS_LAST = N - (NS - 1) * ROWS_MAIN

_mesh = plsc.VectorSubcoreMesh(core_axis_name="c", subcore_axis_name="s")


def _per_tile_rows(s, fn):
    r0 = pl.multiple_of(s * ROWS_MAIN, 8)

    @pl.when(s < NS - 1)
    def _():
        fn(r0, ROWS_MAIN)

    @pl.when(s == NS - 1)
    def _():
        fn((NS - 1) * ROWS_MAIN, ROWS_LAST)


def _deg_body(dst_hbm, ones_hbm, zeros_hbm, out_hbm,
              didx, didx_t, ones_v, ones_t, acc, isem):
    c = lax.axis_index("c")
    s = lax.axis_index("s")
    wid = c * NS + s

    _per_tile_rows(s, lambda r0, nr: pltpu.sync_copy(
        zeros_hbm.at[pl.ds(r0, nr)], acc.at[pl.ds(r0, nr)]))
    pltpu.sync_copy(ones_hbm, ones_v)
    pltpu.sync_copy(ones_hbm.at[pl.ds(0, DEG_TAIL)], ones_t)
    plsc.subcore_barrier()

    base = wid * EDGES_PER_WORKER

    def idx_copy(b, off):
        e0 = base + b * IDX_BLK
        return pltpu.make_async_copy(dst_hbm.at[pl.ds(e0, IDX_BLK)],
                                     didx.at[pl.ds(off, IDX_BLK)], isem)

    idx_copy(0, 0).start()

    def blk(b, _):
        off_cur = (b % 2) * IDX_BLK
        off_nxt = ((b + 1) % 2) * IDX_BLK
        idx_copy(b, off_cur).wait()

        @pl.when(b + 1 < DEG_BLKS)
        def _():
            idx_copy(b + 1, off_nxt).start()

        for k in range(IDX_CHUNKS):
            o = off_cur + k * CHUNK
            pltpu.sync_copy(ones_v, acc.at[didx.at[pl.ds(o, CHUNK)]],
                            add=True)
        return 0
    lax.fori_loop(0, DEG_BLKS, blk, 0)
    if DEG_TAIL:
        e0 = base + DEG_FULL * CHUNK
        pltpu.sync_copy(dst_hbm.at[pl.ds(e0, DEG_TAIL)], didx_t)
        pltpu.sync_copy(ones_t, acc.at[didx_t], add=True)
    plsc.subcore_barrier()

    _per_tile_rows(s, lambda r0, nr: pltpu.sync_copy(
        acc.at[pl.ds(r0, nr)], out_hbm.at[c].at[pl.ds(r0, nr)]))


_deg_kernel = functools.partial(
    pl.kernel,
    out_type=jax.ShapeDtypeStruct((NC, N, D), jnp.float32),
    mesh=_mesh,
    scratch_types=[
        pltpu.VMEM((2 * IDX_BLK,), jnp.int32),
        pltpu.VMEM((DEG_TAIL,), jnp.int32),
        pltpu.VMEM((CHUNK, D), jnp.float32),
        pltpu.VMEM((DEG_TAIL, D), jnp.float32),
        pltpu.VMEM_SHARED((N, D), jnp.float32),
        pltpu.SemaphoreType.DMA,
    ],
)(_deg_body)


NBUF = 2


def _agg_body(feats_hbm, src_hbm, dst_hbm, zeros_hbm, out_hbm, *scr):
    sidx, didx = scr[0], scr[1]
    rowss = scr[2:2 + NBUF]
    sidx_t, didx_t, rows_t, acc = scr[2 + NBUF:6 + NBUF]
    sems = scr[6 + NBUF:6 + 2 * NBUF]
    isems = scr[6 + 2 * NBUF:]

    c = lax.axis_index("c")
    s = lax.axis_index("s")

    _per_tile_rows(s, lambda r0, nr: pltpu.sync_copy(
        zeros_hbm.at[pl.ds(r0, nr)], acc.at[pl.ds(r0, nr)]))
    plsc.subcore_barrier()

    table = feats_hbm.at[c]
    base = s * EDGES_PER_TILE

    def idx_copies(b, off):
        e0 = base + b * AGG_IDXB
        return (
            pltpu.make_async_copy(src_hbm.at[pl.ds(e0, AGG_IDXB)],
                                  sidx.at[pl.ds(off, AGG_IDXB)], isems[0]),
            pltpu.make_async_copy(dst_hbm.at[pl.ds(e0, AGG_IDXB)],
                                  didx.at[pl.ds(off, AGG_IDXB)], isems[1]),
        )

    def idx_issue(b, off):
        for cp in idx_copies(b, off):
            cp.start()

    def idx_wait(b, off):
        for cp in idx_copies(b, off):
            cp.wait()

    def gat(off_elems, r):
        pltpu.async_copy(table.at[sidx.at[pl.ds(off_elems, CHUNK)]],
                         rowss[r], sems[r])

    idx_issue(0, 0)
    idx_wait(0, 0)
    for r in range(NBUF):
        gat(r * CHUNK, r)

    def blk(b, _):
        off_cur = (b % 2) * AGG_IDXB
        off_nxt = ((b + 1) % 2) * AGG_IDXB

        @pl.when(b + 1 < AGG_BLKS)
        def _():
            idx_issue(b + 1, off_nxt)

        for k in range(AGG_IDXC):
            r = k % NBUF
            o = off_cur + k * CHUNK
            pltpu.make_async_copy(table.at[sidx.at[pl.ds(o, CHUNK)]],
                                  rowss[r], sems[r]).wait()
            pltpu.sync_copy(rowss[r], acc.at[didx.at[pl.ds(o, CHUNK)]],
                            add=True)
            kn = k + NBUF
            if kn < AGG_IDXC:
                gat(off_cur + kn * CHUNK, r)
            else:
                @pl.when(b + 1 < AGG_BLKS)
                def _():
                    if kn == AGG_IDXC:
                        idx_wait(b + 1, off_nxt)
                    gat(off_nxt + (kn - AGG_IDXC) * CHUNK, r)
        return 0
    lax.fori_loop(0, AGG_BLKS, blk, 0)

    if TAIL:
        e0 = base + FULL_CHUNKS * CHUNK
        pltpu.sync_copy(src_hbm.at[pl.ds(e0, TAIL)], sidx_t)
        pltpu.sync_copy(dst_hbm.at[pl.ds(e0, TAIL)], didx_t)
        pltpu.async_copy(table.at[sidx_t], rows_t, sems[0]).wait()
        pltpu.sync_copy(rows_t, acc.at[didx_t], add=True)
    plsc.subcore_barrier()

    _per_tile_rows(s, lambda r0, nr: pltpu.sync_copy(
        acc.at[pl.ds(r0, nr)], out_hbm.at[c].at[pl.ds(r0, nr)]))


_agg_kernel = functools.partial(
    pl.kernel,
    out_type=jax.ShapeDtypeStruct((NC, N, D), jnp.float32),
    mesh=_mesh,
    scratch_types=(
        [pltpu.VMEM((2 * AGG_IDXB,), jnp.int32) for _ in range(2)]
        + [pltpu.VMEM((CHUNK, D), jnp.float32) for _ in range(NBUF)]
        + [
            pltpu.VMEM((TAIL,), jnp.int32),
            pltpu.VMEM((TAIL,), jnp.int32),
            pltpu.VMEM((TAIL, D), jnp.float32),
            pltpu.VMEM_SHARED((N, D), jnp.float32),
        ]
        + [pltpu.SemaphoreType.DMA for _ in range(NBUF + 2)]
    ),
)(_agg_body)


BLK = 1000


def _pre_body(degp_ref, x_ref, hx_ref, dis_ref, xnhn_ref):
    deg = degp_ref[0, :, 0:1] + degp_ref[1, :, 0:1]
    dis = lax.rsqrt(jnp.maximum(deg, 1.0))
    dis_ref[...] = dis
    xnhn_ref[0] = x_ref[...] * dis
    xnhn_ref[1] = hx_ref[...] * dis


def _tc_pre(degp, x, hx):
    return pl.pallas_call(
        _pre_body,
        grid=(N // BLK,),
        in_specs=[
            pl.BlockSpec((NC, BLK, D), lambda i: (0, i, 0)),
            pl.BlockSpec((BLK, D), lambda i: (i, 0)),
            pl.BlockSpec((BLK, D), lambda i: (i, 0)),
        ],
        out_specs=[
            pl.BlockSpec((BLK, 1), lambda i: (i, 0)),
            pl.BlockSpec((NC, BLK, D), lambda i: (0, i, 0)),
        ],
        out_shape=[
            jax.ShapeDtypeStruct((N, 1), jnp.float32),
            jax.ShapeDtypeStruct((NC, N, D), jnp.float32),
        ],
    )(degp, x, hx)


def _post_body(agg_ref, dis_ref, x_ref, hx_ref,
               w0x_ref, w1x_ref, w0h_ref, w1h_ref, bx_ref, bh_ref, out_ref):
    dis = dis_ref[...]
    tx1x = agg_ref[0] * (-dis)
    tx1h = agg_ref[1] * (-dis)
    x = x_ref[...]
    hx = hx_ref[...]
    f32 = jnp.float32
    gx = (jnp.dot(x, w0x_ref[...], preferred_element_type=f32)
          + jnp.dot(tx1x, w1x_ref[...], preferred_element_type=f32)
          + bx_ref[...])
    gh = (jnp.dot(hx, w0h_ref[...], preferred_element_type=f32)
          + jnp.dot(tx1h, w1h_ref[...], preferred_element_type=f32)
          + bh_ref[...])
    r = jax.nn.sigmoid(gx[:, 0:D] + gh[:, 0:D])
    u = jax.nn.sigmoid(gx[:, D:2 * D] + gh[:, D:2 * D])
    cg = jax.nn.sigmoid(gx[:, 2 * D:3 * D] + gh[:, 2 * D:3 * D] * r)
    out_ref[...] = u * hx + (1.0 - u) * cg


def _tc_post(agg, dis, x, hx, w0x, w1x, w0h, w1h, bx, bh):
    wspec = pl.BlockSpec((D, 3 * D), lambda i: (0, 0))
    bspec = pl.BlockSpec((1, 3 * D), lambda i: (0, 0))
    return pl.pallas_call(
        _post_body,
        grid=(N // BLK,),
        in_specs=[
            pl.BlockSpec((NC, BLK, D), lambda i: (0, i, 0)),
            pl.BlockSpec((BLK, 1), lambda i: (i, 0)),
            pl.BlockSpec((BLK, D), lambda i: (i, 0)),
            pl.BlockSpec((BLK, D), lambda i: (i, 0)),
            wspec, wspec, wspec, wspec, bspec, bspec,
        ],
        out_specs=pl.BlockSpec((BLK, D), lambda i: (i, 0)),
        out_shape=jax.ShapeDtypeStruct((N, D), jnp.float32),
    )(agg, dis, x, hx, w0x, w1x, w0h, w1h, bx, bh)


@jax.jit
def kernel(x, hx, edge_index,
           W0_r_x, W1_r_x, b_r_x, W0_r_h, W1_r_h, b_r_h,
           W0_u_x, W1_u_x, b_u_x, W0_u_h, W1_u_h, b_u_h,
           W0_c_x, W1_c_x, b_c_x, W0_c_h, W1_c_h, b_c_h):
    src = edge_index[0]
    dst = edge_index[1]

    w0x = jnp.concatenate([W0_r_x, W0_u_x, W0_c_x], axis=1)
    w1x = jnp.concatenate([W1_r_x, W1_u_x, W1_c_x], axis=1)
    w0h = jnp.concatenate([W0_r_h, W0_u_h, W0_c_h], axis=1)
    w1h = jnp.concatenate([W1_r_h, W1_u_h, W1_c_h], axis=1)
    bx = jnp.concatenate([b_r_x, b_u_x, b_c_x])[None, :]
    bh = jnp.concatenate([b_r_h, b_u_h, b_c_h])[None, :]

    ones_feat = jnp.ones((CHUNK, D), jnp.float32)
    zeros_feat = jnp.zeros((N, D), jnp.float32)

    degp = _deg_kernel(dst, ones_feat, zeros_feat)
    dis, xnhn = _tc_pre(degp, x, hx)
    agg = _agg_kernel(xnhn, src, dst, zeros_feat)
    return _tc_post(agg, dis, x, hx, w0x, w1x, w0h, w1h, bx, bh)
